# baseline (device time: 333053 ns/iter reference)
import os

import jax
import jax.numpy as jnp
from jax import lax
from jax.experimental import pallas as pl
from jax.experimental.pallas import tpu as pltpu

N_DEV = 8
SQ = 256
DM = 1024
HQL = 8
HG = 64
DH = 128
NG = 16
RES = 4
KB = 64
SCALE = 0.08838834764831843

_BITS = (4, 2, 1)
_RB_OFF = (0, 128, 192)

_SKIP_AR = bool(int(os.environ.get("SKIP_AR", "0")))


def kernel(x, Wq, K_ext, V_ext, Wo):
    x2 = x.reshape(SQ, DM)
    K3 = K_ext.reshape(NG, RES, KB, HG * DH)
    V3 = V_ext.reshape(NG, RES, KB, HG * DH)

    def body(x_ref, wq_ref, k_hbm, v_hbm, wo_ref, out_ref,
             kbuf, vbuf, ctx_buf, pbuf, rbuf,
             ksems, vsems, rs_send, rs_recv, ag_send, ag_recv):
        my = lax.axis_index("i")
        c0 = my * (HQL * DH)

        kcopies = []
        vcopies = []
        for qb in range(RES):
            kc = pltpu.make_async_copy(
                k_hbm.at[:, qb, :, pl.ds(c0, HQL * DH)], kbuf.at[qb],
                ksems.at[qb])
            vc = pltpu.make_async_copy(
                v_hbm.at[:, qb, :, pl.ds(c0, HQL * DH)], vbuf.at[qb],
                vsems.at[qb])
            kc.start()
            vc.start()
            kcopies.append(kc)
            vcopies.append(vc)

        q = jnp.dot(x_ref[:, :], wq_ref[:, :],
                    preferred_element_type=jnp.float32) * SCALE

        for qb in range(RES):
            kcopies[qb].wait()
            vcopies[qb].wait()
            for h in range(HQL):
                kv_k = kbuf[qb, :, :, h * DH:(h + 1) * DH].reshape(NG * KB, DH)
                kv_v = vbuf[qb, :, :, h * DH:(h + 1) * DH].reshape(NG * KB, DH)
                qh = q[qb * KB:(qb + 1) * KB, h * DH:(h + 1) * DH]
                s = lax.dot_general(
                    qh, kv_k, (((1,), (1,)), ((), ())),
                    preferred_element_type=jnp.float32,
                )
                m = jnp.max(s, axis=1, keepdims=True)
                e = jnp.exp(s - m)
                p = e / jnp.sum(e, axis=1, keepdims=True)
                ctx = jnp.dot(p, kv_v, preferred_element_type=jnp.float32)
                ctx_buf[qb * KB:(qb + 1) * KB, h * DH:(h + 1) * DH] = ctx

        partial = jnp.dot(ctx_buf[:, :], wo_ref[:, :],
                          preferred_element_type=jnp.float32)
        pbuf[:, :] = partial.astype(jnp.bfloat16)

        if _SKIP_AR:
            out_ref[0] = pbuf[:, :].astype(jnp.float32)
            return

        barrier_sem = pltpu.get_barrier_semaphore()
        for b in _BITS:
            pl.semaphore_signal(
                barrier_sem, inc=1,
                device_id=(my ^ b,), device_id_type=pl.DeviceIdType.MESH,
            )
        pl.semaphore_wait(barrier_sem, len(_BITS))

        seg_start = jnp.int32(0)
        for r, b in enumerate(_BITS):
            half = SQ >> (r + 1)
            partner = my ^ b
            mybit = (my & b) != 0
            send_off = seg_start + jnp.where(mybit, 0, half)
            keep_off = seg_start + jnp.where(mybit, half, 0)
            rdma = pltpu.make_async_remote_copy(
                src_ref=pbuf.at[pl.ds(send_off, half), :],
                dst_ref=rbuf.at[pl.ds(_RB_OFF[r], half), :],
                send_sem=rs_send.at[r],
                recv_sem=rs_recv.at[r],
                device_id=(partner,),
                device_id_type=pl.DeviceIdType.MESH,
            )
            rdma.start()
            rdma.wait()
            acc = (pbuf[pl.ds(keep_off, half), :].astype(jnp.float32)
                   + rbuf[pl.ds(_RB_OFF[r], half), :].astype(jnp.float32))
            pbuf[pl.ds(keep_off, half), :] = acc.astype(jnp.bfloat16)
            seg_start = keep_off

        for r2, b in enumerate(_BITS[::-1]):
            ln = 32 << r2
            partner = my ^ b
            own_start = 32 * ((my >> r2) << r2)
            rdma = pltpu.make_async_remote_copy(
                src_ref=pbuf.at[pl.ds(own_start, ln), :],
                dst_ref=pbuf.at[pl.ds(own_start, ln), :],
                send_sem=ag_send.at[r2],
                recv_sem=ag_recv.at[r2],
                device_id=(partner,),
                device_id_type=pl.DeviceIdType.MESH,
            )
            rdma.start()
            rdma.wait()

        out_ref[0] = pbuf[:, :].astype(jnp.float32)

    return pl.pallas_call(
        body,
        out_shape=jax.ShapeDtypeStruct((1, SQ, DM), jnp.float32),
        in_specs=[
            pl.BlockSpec(memory_space=pltpu.VMEM),
            pl.BlockSpec(memory_space=pltpu.VMEM),
            pl.BlockSpec(memory_space=pl.ANY),
            pl.BlockSpec(memory_space=pl.ANY),
            pl.BlockSpec(memory_space=pltpu.VMEM),
        ],
        out_specs=pl.BlockSpec(memory_space=pltpu.VMEM),
        scratch_shapes=[
            pltpu.VMEM((RES, NG, KB, HQL * DH), jnp.float32),
            pltpu.VMEM((RES, NG, KB, HQL * DH), jnp.float32),
            pltpu.VMEM((SQ, DM), jnp.float32),
            pltpu.VMEM((SQ, DM), jnp.bfloat16),
            pltpu.VMEM((224, DM), jnp.bfloat16),
            pltpu.SemaphoreType.DMA((RES,)),
            pltpu.SemaphoreType.DMA((RES,)),
            pltpu.SemaphoreType.DMA((3,)),
            pltpu.SemaphoreType.DMA((3,)),
            pltpu.SemaphoreType.DMA((3,)),
            pltpu.SemaphoreType.DMA((3,)),
        ],
        compiler_params=pltpu.CompilerParams(
            collective_id=None if _SKIP_AR else 0,
            vmem_limit_bytes=110 * 1024 * 1024,
        ),
    )(x2, Wq, K3, V3, Wo)


# device time: 52618 ns/iter; 6.3296x vs baseline; 6.3296x over previous
import os

import jax
import jax.numpy as jnp
from jax import lax
from jax.experimental import pallas as pl
from jax.experimental.pallas import tpu as pltpu

N_DEV = 8
SQ = 256
DM = 1024
HQL = 8
HG = 64
DH = 128
NG = 16
RES = 4
KB = 64
SCALE = 0.08838834764831843

_BITS = (4, 2, 1)
_RB_OFF = (0, 128, 192)

_SKIP_AR = bool(int(os.environ.get("SKIP_AR", "0")))


def kernel(x, Wq, K_ext, V_ext, Wo):
    x2 = x.reshape(SQ, DM)
    K3 = K_ext.reshape(NG, RES, KB, HG, DH)
    V3 = V_ext.reshape(NG, RES, KB, HG, DH)

    def body(x_ref, wq_ref, k_hbm, v_hbm, wo_ref, out_ref,
             kbuf, vbuf, ctx_buf, pbuf, rbuf,
             ksems, vsems, rs_send, rs_recv, ag_send, ag_recv):
        my = lax.axis_index("i")
        h0 = my * HQL

        kcopies = {}
        vcopies = {}
        for qb in range(RES):
            for h in range(HQL):
                hg = h0 + h
                kc = pltpu.make_async_copy(
                    k_hbm.at[:, qb, :, hg, :], kbuf.at[qb, h], ksems.at[qb, h])
                vc = pltpu.make_async_copy(
                    v_hbm.at[:, qb, :, hg, :], vbuf.at[qb, h], vsems.at[qb, h])
                kc.start()
                vc.start()
                kcopies[qb, h] = kc
                vcopies[qb, h] = vc

        q = jnp.dot(x_ref[:, :], wq_ref[:, :],
                    preferred_element_type=jnp.float32) * SCALE

        for qb in range(RES):
            for h in range(HQL):
                kcopies[qb, h].wait()
                vcopies[qb, h].wait()
                kv_k = kbuf[qb, h].reshape(NG * KB, DH)
                kv_v = vbuf[qb, h].reshape(NG * KB, DH)
                qh = q[qb * KB:(qb + 1) * KB, h * DH:(h + 1) * DH]
                s = lax.dot_general(
                    qh, kv_k, (((1,), (1,)), ((), ())),
                    preferred_element_type=jnp.float32,
                )
                m = jnp.max(s, axis=1, keepdims=True)
                e = jnp.exp(s - m)
                p = e / jnp.sum(e, axis=1, keepdims=True)
                ctx = jnp.dot(p, kv_v, preferred_element_type=jnp.float32)
                ctx_buf[qb * KB:(qb + 1) * KB, h * DH:(h + 1) * DH] = ctx

        partial = jnp.dot(ctx_buf[:, :], wo_ref[:, :],
                          preferred_element_type=jnp.float32)
        pbuf[:, :] = partial.astype(jnp.bfloat16)

        if _SKIP_AR:
            out_ref[0] = pbuf[:, :].astype(jnp.float32)
            return

        barrier_sem = pltpu.get_barrier_semaphore()
        for b in _BITS:
            pl.semaphore_signal(
                barrier_sem, inc=1,
                device_id=(my ^ b,), device_id_type=pl.DeviceIdType.MESH,
            )
        pl.semaphore_wait(barrier_sem, len(_BITS))

        seg_start = jnp.int32(0)
        for r, b in enumerate(_BITS):
            half = SQ >> (r + 1)
            partner = my ^ b
            mybit = (my & b) != 0
            send_off = seg_start + jnp.where(mybit, 0, half)
            keep_off = seg_start + jnp.where(mybit, half, 0)
            rdma = pltpu.make_async_remote_copy(
                src_ref=pbuf.at[pl.ds(send_off, half), :],
                dst_ref=rbuf.at[pl.ds(_RB_OFF[r], half), :],
                send_sem=rs_send.at[r],
                recv_sem=rs_recv.at[r],
                device_id=(partner,),
                device_id_type=pl.DeviceIdType.MESH,
            )
            rdma.start()
            rdma.wait()
            acc = (pbuf[pl.ds(keep_off, half), :].astype(jnp.float32)
                   + rbuf[pl.ds(_RB_OFF[r], half), :].astype(jnp.float32))
            pbuf[pl.ds(keep_off, half), :] = acc.astype(jnp.bfloat16)
            seg_start = keep_off

        for r2, b in enumerate(_BITS[::-1]):
            ln = 32 << r2
            partner = my ^ b
            own_start = 32 * ((my >> r2) << r2)
            rdma = pltpu.make_async_remote_copy(
                src_ref=pbuf.at[pl.ds(own_start, ln), :],
                dst_ref=pbuf.at[pl.ds(own_start, ln), :],
                send_sem=ag_send.at[r2],
                recv_sem=ag_recv.at[r2],
                device_id=(partner,),
                device_id_type=pl.DeviceIdType.MESH,
            )
            rdma.start()
            rdma.wait()

        out_ref[0] = pbuf[:, :].astype(jnp.float32)

    return pl.pallas_call(
        body,
        out_shape=jax.ShapeDtypeStruct((1, SQ, DM), jnp.float32),
        in_specs=[
            pl.BlockSpec(memory_space=pltpu.VMEM),
            pl.BlockSpec(memory_space=pltpu.VMEM),
            pl.BlockSpec(memory_space=pl.ANY),
            pl.BlockSpec(memory_space=pl.ANY),
            pl.BlockSpec(memory_space=pltpu.VMEM),
        ],
        out_specs=pl.BlockSpec(memory_space=pltpu.VMEM),
        scratch_shapes=[
            pltpu.VMEM((RES, HQL, NG, KB, DH), jnp.float32),
            pltpu.VMEM((RES, HQL, NG, KB, DH), jnp.float32),
            pltpu.VMEM((SQ, DM), jnp.float32),
            pltpu.VMEM((SQ, DM), jnp.bfloat16),
            pltpu.VMEM((224, DM), jnp.bfloat16),
            pltpu.SemaphoreType.DMA((RES, HQL)),
            pltpu.SemaphoreType.DMA((RES, HQL)),
            pltpu.SemaphoreType.DMA((3,)),
            pltpu.SemaphoreType.DMA((3,)),
            pltpu.SemaphoreType.DMA((3,)),
            pltpu.SemaphoreType.DMA((3,)),
        ],
        compiler_params=pltpu.CompilerParams(
            collective_id=None if _SKIP_AR else 0,
            vmem_limit_bytes=110 * 1024 * 1024,
        ),
    )(x2, Wq, K3, V3, Wo)


# device time: 20954 ns/iter; 15.8945x vs baseline; 2.5111x over previous
import os

import jax
import jax.numpy as jnp
from jax import lax
from jax.experimental import pallas as pl
from jax.experimental.pallas import tpu as pltpu

N_DEV = 8
SQ = 256
DM = 1024
HQL = 8
HG = 64
DH = 128
NG = 16
RES = 4
KB = 64
SCALE = 0.08838834764831843

_BITS = (4, 2, 1)
_RB_OFF = (0, 128, 192)

_SKIP_AR = bool(int(os.environ.get("SKIP_AR", "0")))
_SKIP_DMA = bool(int(os.environ.get("SKIP_DMA", "0")))
_SKIP_ATTN = bool(int(os.environ.get("SKIP_ATTN", "0")))


def kernel(x, Wq, K_ext, V_ext, Wo):
    x2 = x.reshape(SQ, DM)
    K3 = K_ext.reshape(NG, RES, KB, HG, DH)
    V3 = V_ext.reshape(NG, RES, KB, HG, DH)

    def body(x_ref, wq_ref, k_hbm, v_hbm, wo_ref, out_ref,
             kbuf, vbuf, ctx_buf, pbuf, rbuf,
             ksems, vsems, rs_send, rs_recv, ag_send, ag_recv):
        my = lax.axis_index("i")
        h0 = my * HQL

        kcopies = {}
        vcopies = {}
        if not _SKIP_DMA:
            for qb in range(RES):
                for h in range(HQL):
                    hg = h0 + h
                    kc = pltpu.make_async_copy(
                        k_hbm.at[:, qb, :, hg, :], kbuf.at[qb, h],
                        ksems.at[qb, h])
                    vc = pltpu.make_async_copy(
                        v_hbm.at[:, qb, :, hg, :], vbuf.at[qb, h],
                        vsems.at[qb, h])
                    kc.start()
                    vc.start()
                    kcopies[qb, h] = kc
                    vcopies[qb, h] = vc

        q = jnp.dot(x_ref[:, :], wq_ref[:, :],
                    preferred_element_type=jnp.float32) * SCALE

        for qb in range(RES):
            for h in range(HQL):
                if not _SKIP_DMA:
                    kcopies[qb, h].wait()
                    vcopies[qb, h].wait()
                if _SKIP_ATTN:
                    continue
                kv_k = kbuf[qb, h].reshape(NG * KB, DH)
                kv_v = vbuf[qb, h].reshape(NG * KB, DH)
                qh = q[qb * KB:(qb + 1) * KB, h * DH:(h + 1) * DH]
                s = lax.dot_general(
                    qh, kv_k, (((1,), (1,)), ((), ())),
                    preferred_element_type=jnp.float32,
                )
                m = jnp.max(s, axis=1, keepdims=True)
                e = jnp.exp(s - m)
                p = e / jnp.sum(e, axis=1, keepdims=True)
                ctx = jnp.dot(p, kv_v, preferred_element_type=jnp.float32)
                ctx_buf[qb * KB:(qb + 1) * KB, h * DH:(h + 1) * DH] = ctx

        partial = jnp.dot(ctx_buf[:, :], wo_ref[:, :],
                          preferred_element_type=jnp.float32)
        pbuf[:, :] = partial.astype(jnp.bfloat16)

        if _SKIP_AR:
            out_ref[0] = pbuf[:, :].astype(jnp.float32)
            return

        barrier_sem = pltpu.get_barrier_semaphore()
        for b in _BITS:
            pl.semaphore_signal(
                barrier_sem, inc=1,
                device_id=(my ^ b,), device_id_type=pl.DeviceIdType.MESH,
            )
        pl.semaphore_wait(barrier_sem, len(_BITS))

        seg_start = jnp.int32(0)
        for r, b in enumerate(_BITS):
            half = SQ >> (r + 1)
            partner = my ^ b
            mybit = (my & b) != 0
            send_off = seg_start + jnp.where(mybit, 0, half)
            keep_off = seg_start + jnp.where(mybit, half, 0)
            rdma = pltpu.make_async_remote_copy(
                src_ref=pbuf.at[pl.ds(send_off, half), :],
                dst_ref=rbuf.at[pl.ds(_RB_OFF[r], half), :],
                send_sem=rs_send.at[r],
                recv_sem=rs_recv.at[r],
                device_id=(partner,),
                device_id_type=pl.DeviceIdType.MESH,
            )
            rdma.start()
            rdma.wait()
            acc = (pbuf[pl.ds(keep_off, half), :].astype(jnp.float32)
                   + rbuf[pl.ds(_RB_OFF[r], half), :].astype(jnp.float32))
            pbuf[pl.ds(keep_off, half), :] = acc.astype(jnp.bfloat16)
            seg_start = keep_off

        for r2, b in enumerate(_BITS[::-1]):
            ln = 32 << r2
            partner = my ^ b
            own_start = 32 * ((my >> r2) << r2)
            rdma = pltpu.make_async_remote_copy(
                src_ref=pbuf.at[pl.ds(own_start, ln), :],
                dst_ref=pbuf.at[pl.ds(own_start, ln), :],
                send_sem=ag_send.at[r2],
                recv_sem=ag_recv.at[r2],
                device_id=(partner,),
                device_id_type=pl.DeviceIdType.MESH,
            )
            rdma.start()
            rdma.wait()

        out_ref[0] = pbuf[:, :].astype(jnp.float32)

    return pl.pallas_call(
        body,
        out_shape=jax.ShapeDtypeStruct((1, SQ, DM), jnp.float32),
        in_specs=[
            pl.BlockSpec(memory_space=pltpu.VMEM),
            pl.BlockSpec(memory_space=pltpu.VMEM),
            pl.BlockSpec(memory_space=pl.ANY),
            pl.BlockSpec(memory_space=pl.ANY),
            pl.BlockSpec(memory_space=pltpu.VMEM),
        ],
        out_specs=pl.BlockSpec(memory_space=pltpu.VMEM),
        scratch_shapes=[
            pltpu.VMEM((RES, HQL, NG, KB, DH), jnp.float32),
            pltpu.VMEM((RES, HQL, NG, KB, DH), jnp.float32),
            pltpu.VMEM((SQ, DM), jnp.float32),
            pltpu.VMEM((SQ, DM), jnp.bfloat16),
            pltpu.VMEM((224, DM), jnp.bfloat16),
            pltpu.SemaphoreType.DMA((RES, HQL)),
            pltpu.SemaphoreType.DMA((RES, HQL)),
            pltpu.SemaphoreType.DMA((3,)),
            pltpu.SemaphoreType.DMA((3,)),
            pltpu.SemaphoreType.DMA((3,)),
            pltpu.SemaphoreType.DMA((3,)),
        ],
        compiler_params=pltpu.CompilerParams(
            collective_id=None if _SKIP_AR else 0,
            vmem_limit_bytes=110 * 1024 * 1024,
        ),
    )(x2, Wq, K3, V3, Wo)
